# baseline (speedup 1.0000x reference)
import functools

import jax
import jax.numpy as jnp
from jax import lax
from jax.experimental import pallas as pl
from jax.experimental.pallas import tpu as pltpu
from jax.experimental.pallas import tpu_sc as plsc

N = 10000
NP = 10240
C = 128
E = 320000
NC, NS = 2, 16
NW = NC * NS
EPW = E // NW
CHUNK = 80
NCH = EPW // CHUNK
RPT = NP // NS
NB = 5
PB = 2

_mesh = plsc.VectorSubcoreMesh(
    core_axis_name="c", subcore_axis_name="s", num_cores=NC, num_subcores=NS)


@functools.partial(
    pl.kernel,
    out_type=jax.ShapeDtypeStruct((NC, 1, NP), jnp.float32),
    mesh=_mesh,
    scratch_types=[
        pltpu.VMEM((NCH, CHUNK), jnp.int32),
        pltpu.VMEM((CHUNK,), jnp.float32),
        pltpu.VMEM_SHARED((NP,), jnp.float32),
    ] + [pltpu.SemaphoreType.DMA] * NB,
)
def _deg_kernel(dst3, zeros_hbm, ones_hbm, out, dstbuf, onesv, degsh,
                *sems):
    c = lax.axis_index("c")
    s = lax.axis_index("s")
    w = c * NS + s
    pltpu.sync_copy(dst3.at[w], dstbuf)
    pltpu.sync_copy(ones_hbm, onesv)

    @pl.when(s == 0)
    def _():
        pltpu.sync_copy(zeros_hbm, degsh)

    plsc.subcore_barrier()

    def body(g, carry):
        for b in range(NB):
            j = g * NB + b

            @pl.when(g > 0)
            def _():
                pltpu.make_async_copy(
                    onesv, degsh.at[dstbuf.at[j]], sems[b]).wait()

            pltpu.async_copy(onesv, degsh.at[dstbuf.at[j]], sems[b],
                             add=True)
        return carry

    lax.fori_loop(0, NCH // NB, body, 0)
    for b in range(NB):
        pltpu.make_async_copy(onesv, degsh.at[dstbuf.at[0]], sems[b]).wait()
    plsc.subcore_barrier()
    pltpu.sync_copy(degsh.at[pl.ds(s * RPT, RPT)],
                    out.at[c, 0, pl.ds(s * RPT, RPT)])


@functools.partial(
    pl.kernel,
    out_type=jax.ShapeDtypeStruct((NC, NP, C), jnp.float32),
    mesh=_mesh,
    scratch_types=[
        pltpu.VMEM((EPW,), jnp.int32),
        pltpu.VMEM((EPW,), jnp.int32),
        pltpu.VMEM((PB * CHUNK, C), jnp.float32),
        pltpu.VMEM_SHARED((NP, C), jnp.float32),
    ] + [pltpu.SemaphoreType.DMA] * PB,
)
def _prop_kernel(p, src2, dst2, out, srcbuf, dstbuf, rows, accsh, *gsem):
    c = lax.axis_index("c")
    s = lax.axis_index("s")
    w = c * NS + s
    pltpu.sync_copy(src2.at[w], srcbuf)
    pltpu.sync_copy(dst2.at[w], dstbuf)

    def slot(b):
        return rows.at[pl.ds(b * CHUNK, CHUNK)]

    def sidx(j):
        return srcbuf.at[pl.ds(j * CHUNK, CHUNK)]

    def didx(j):
        return dstbuf.at[pl.ds(j * CHUNK, CHUNK)]

    for b in range(PB):
        pltpu.async_copy(p.at[sidx(b)], slot(b), gsem[b])

    base = s * RPT
    pltpu.sync_copy(p.at[pl.ds(base, RPT)], accsh.at[pl.ds(base, RPT)])
    plsc.subcore_barrier()

    def body(g, carry):
        for b in range(PB):
            j = g * PB + b
            pltpu.make_async_copy(p.at[sidx(j)], slot(b), gsem[b]).wait()
            pltpu.sync_copy(slot(b), accsh.at[didx(j)], add=True)

            @pl.when(j + PB < NCH)
            def _():
                pltpu.async_copy(p.at[sidx(j + PB)], slot(b), gsem[b])
        return carry

    lax.fori_loop(0, NCH // PB, body, 0)
    for j in range(PB * (NCH // PB), NCH):
        eb = j % PB
        pltpu.make_async_copy(p.at[sidx(j)], slot(eb), gsem[eb]).wait()
        pltpu.sync_copy(slot(eb), accsh.at[didx(j)], add=True)
    plsc.subcore_barrier()
    pltpu.sync_copy(accsh.at[pl.ds(base, RPT)], out.at[c, pl.ds(base, RPT)])


_RB = 2048


def _prep_body(deg_ref, x_ref, w1_ref, p1_ref, dis_ref):
    deg = deg_ref[0] + deg_ref[1] + 1.0
    dis = lax.rsqrt(deg)
    t = jax.lax.dot_general(
        x_ref[...], w1_ref[...], (((1,), (0,)), ((), ())),
        preferred_element_type=jnp.float32,
        precision=jax.lax.Precision.HIGHEST)
    p1_ref[...] = t * dis
    dis_ref[...] = dis


def _tc_prep(deg3, x, W1):
    return pl.pallas_call(
        _prep_body,
        grid=(NP // _RB,),
        in_specs=[
            pl.BlockSpec((NC, _RB, 1), lambda i: (0, i, 0)),
            pl.BlockSpec((_RB, C), lambda i: (i, 0)),
            pl.BlockSpec((C, C), lambda i: (0, 0)),
        ],
        out_specs=[
            pl.BlockSpec((_RB, C), lambda i: (i, 0)),
            pl.BlockSpec((_RB, 1), lambda i: (i, 0)),
        ],
        out_shape=[
            jax.ShapeDtypeStruct((NP, C), jnp.float32),
            jax.ShapeDtypeStruct((NP, 1), jnp.float32),
        ],
    )(deg3, x, W1)


def _mid_body(acc_ref, p1_ref, dis_ref, b1_ref, w2_ref, p2_ref):
    dis = dis_ref[...]
    agg = acc_ref[0] + acc_ref[1] - p1_ref[...]
    h = jnp.maximum(agg * dis + b1_ref[...], 0.0)
    t = jax.lax.dot_general(
        h, w2_ref[...], (((1,), (0,)), ((), ())),
        preferred_element_type=jnp.float32,
        precision=jax.lax.Precision.HIGHEST)
    p2_ref[...] = t * dis


def _tc_mid(acc, p1, dis, b1, W2):
    return pl.pallas_call(
        _mid_body,
        grid=(NP // _RB,),
        in_specs=[
            pl.BlockSpec((NC, _RB, C), lambda i: (0, i, 0)),
            pl.BlockSpec((_RB, C), lambda i: (i, 0)),
            pl.BlockSpec((_RB, 1), lambda i: (i, 0)),
            pl.BlockSpec((1, C), lambda i: (0, 0)),
            pl.BlockSpec((C, C), lambda i: (0, 0)),
        ],
        out_specs=pl.BlockSpec((_RB, C), lambda i: (i, 0)),
        out_shape=jax.ShapeDtypeStruct((NP, C), jnp.float32),
    )(acc, p1, dis, b1, W2)


def _final_body(acc_ref, p2_ref, dis_ref, b2_ref, z_ref):
    agg = acc_ref[0] + acc_ref[1] - p2_ref[...]
    z_ref[...] = agg * dis_ref[...] + b2_ref[...]


def _tc_final(acc, p2, dis, b2):
    return pl.pallas_call(
        _final_body,
        grid=(NP // _RB,),
        in_specs=[
            pl.BlockSpec((NC, _RB, C), lambda i: (0, i, 0)),
            pl.BlockSpec((_RB, C), lambda i: (i, 0)),
            pl.BlockSpec((_RB, 1), lambda i: (i, 0)),
            pl.BlockSpec((1, C), lambda i: (0, 0)),
        ],
        out_specs=pl.BlockSpec((_RB, C), lambda i: (i, 0)),
        out_shape=jax.ShapeDtypeStruct((NP, C), jnp.float32),
    )(acc, p2, dis, b2)


def kernel(x, edge_index, W1, b1, W2, b2):
    src2 = edge_index[0].reshape(NW, EPW)
    dst2 = edge_index[1].reshape(NW, EPW)
    dst3 = dst2.reshape(NW, NCH, CHUNK)
    zeros = jnp.zeros((NP,), jnp.float32)
    ones = jnp.ones((CHUNK,), jnp.float32)
    x_p = jnp.pad(x, ((0, NP - N), (0, 0)))

    deg_p = _deg_kernel(dst3, zeros, ones)
    deg3 = deg_p.reshape(NC, NP, 1)
    p1, dis = _tc_prep(deg3, x_p, W1)
    acc1 = _prop_kernel(p1, src2, dst2)
    p2 = _tc_mid(acc1, p1, dis, b1.reshape(1, C), W2)
    acc2 = _prop_kernel(p2, src2, dst2)
    z = _tc_final(acc2, p2, dis, b2.reshape(1, C))
    return z[:N]


# speedup vs baseline: 1.0117x; 1.0117x over previous
import functools

import jax
import jax.numpy as jnp
from jax import lax
from jax.experimental import pallas as pl
from jax.experimental.pallas import tpu as pltpu
from jax.experimental.pallas import tpu_sc as plsc

N = 10000
NP = 10240
C = 128
E = 320000
NC, NS = 2, 16
NW = NC * NS
EPW = E // NW
CHUNK = 80
NCH = EPW // CHUNK
RPT = NP // NS
NB = 5
PB = 2

_mesh = plsc.VectorSubcoreMesh(
    core_axis_name="c", subcore_axis_name="s", num_cores=NC, num_subcores=NS)


@functools.partial(
    pl.kernel,
    out_type=jax.ShapeDtypeStruct((NC, 1, NP), jnp.float32),
    mesh=_mesh,
    scratch_types=[
        pltpu.VMEM((NCH, CHUNK), jnp.int32),
        pltpu.VMEM((CHUNK,), jnp.float32),
        pltpu.VMEM_SHARED((NP,), jnp.float32),
    ] + [pltpu.SemaphoreType.DMA] * NB,
)
def _deg_kernel(dst3, zeros_hbm, ones_hbm, out, dstbuf, onesv, degsh,
                *sems):
    c = lax.axis_index("c")
    s = lax.axis_index("s")
    w = c * NS + s
    pltpu.sync_copy(dst3.at[w], dstbuf)
    pltpu.sync_copy(ones_hbm, onesv)

    @pl.when(s == 0)
    def _():
        pltpu.sync_copy(zeros_hbm, degsh)

    plsc.subcore_barrier()

    def body(g, carry):
        for b in range(NB):
            j = g * NB + b

            @pl.when(g > 0)
            def _():
                pltpu.make_async_copy(
                    onesv, degsh.at[dstbuf.at[j]], sems[b]).wait()

            pltpu.async_copy(onesv, degsh.at[dstbuf.at[j]], sems[b],
                             add=True)
        return carry

    lax.fori_loop(0, NCH // NB, body, 0)
    for b in range(NB):
        pltpu.make_async_copy(onesv, degsh.at[dstbuf.at[0]], sems[b]).wait()
    plsc.subcore_barrier()
    pltpu.sync_copy(degsh.at[pl.ds(s * RPT, RPT)],
                    out.at[c, 0, pl.ds(s * RPT, RPT)])


@functools.partial(
    pl.kernel,
    out_type=jax.ShapeDtypeStruct((NC, NP, C), jnp.float32),
    mesh=_mesh,
    scratch_types=[
        pltpu.VMEM((EPW,), jnp.int32),
        pltpu.VMEM((EPW,), jnp.int32),
        pltpu.VMEM((PB * CHUNK, C), jnp.float32),
        pltpu.VMEM_SHARED((NP, C), jnp.float32),
    ] + [pltpu.SemaphoreType.DMA] * PB,
)
def _prop_kernel(p, src2, dst2, out, srcbuf, dstbuf, rows, accsh, *gsem):
    c = lax.axis_index("c")
    s = lax.axis_index("s")
    w = c * NS + s
    pltpu.sync_copy(src2.at[w], srcbuf)
    pltpu.sync_copy(dst2.at[w], dstbuf)

    def slot(b):
        return rows.at[pl.ds(b * CHUNK, CHUNK)]

    def sidx(j):
        return srcbuf.at[pl.ds(j * CHUNK, CHUNK)]

    def didx(j):
        return dstbuf.at[pl.ds(j * CHUNK, CHUNK)]

    for b in range(PB):
        pltpu.async_copy(p.at[sidx(b)], slot(b), gsem[b])

    base = s * RPT
    pltpu.sync_copy(p.at[pl.ds(base, RPT)], accsh.at[pl.ds(base, RPT)])
    plsc.subcore_barrier()

    def body(g, carry):
        for b in range(PB):
            j = g * PB + b
            pltpu.make_async_copy(p.at[sidx(j)], slot(b), gsem[b]).wait()
            pltpu.sync_copy(slot(b), accsh.at[didx(j)], add=True)

            @pl.when(j + PB < NCH)
            def _():
                pltpu.async_copy(p.at[sidx(j + PB)], slot(b), gsem[b])
        return carry

    lax.fori_loop(0, NCH // PB, body, 0)
    for j in range(PB * (NCH // PB), NCH):
        eb = j % PB
        pltpu.make_async_copy(p.at[sidx(j)], slot(eb), gsem[eb]).wait()
        pltpu.sync_copy(slot(eb), accsh.at[didx(j)], add=True)
    plsc.subcore_barrier()
    pltpu.sync_copy(accsh.at[pl.ds(base, RPT)], out.at[c, pl.ds(base, RPT)])


_RB = 5120


def _prep_body(deg_ref, x_ref, w1_ref, p1_ref, dis_ref):
    deg = deg_ref[0] + deg_ref[1] + 1.0
    dis = lax.rsqrt(deg)
    t = jax.lax.dot_general(
        x_ref[...], w1_ref[...], (((1,), (0,)), ((), ())),
        preferred_element_type=jnp.float32,
        precision=jax.lax.Precision.HIGHEST)
    p1_ref[...] = t * dis
    dis_ref[...] = dis


def _tc_prep(deg3, x, W1):
    return pl.pallas_call(
        _prep_body,
        grid=(NP // _RB,),
        in_specs=[
            pl.BlockSpec((NC, _RB, 1), lambda i: (0, i, 0)),
            pl.BlockSpec((_RB, C), lambda i: (i, 0)),
            pl.BlockSpec((C, C), lambda i: (0, 0)),
        ],
        out_specs=[
            pl.BlockSpec((_RB, C), lambda i: (i, 0)),
            pl.BlockSpec((_RB, 1), lambda i: (i, 0)),
        ],
        out_shape=[
            jax.ShapeDtypeStruct((NP, C), jnp.float32),
            jax.ShapeDtypeStruct((NP, 1), jnp.float32),
        ],
    )(deg3, x, W1)


def _mid_body(acc_ref, p1_ref, dis_ref, b1_ref, w2_ref, p2_ref):
    dis = dis_ref[...]
    agg = acc_ref[0] + acc_ref[1] - p1_ref[...]
    h = jnp.maximum(agg * dis + b1_ref[...], 0.0)
    t = jax.lax.dot_general(
        h, w2_ref[...], (((1,), (0,)), ((), ())),
        preferred_element_type=jnp.float32,
        precision=jax.lax.Precision.HIGHEST)
    p2_ref[...] = t * dis


def _tc_mid(acc, p1, dis, b1, W2):
    return pl.pallas_call(
        _mid_body,
        grid=(NP // _RB,),
        in_specs=[
            pl.BlockSpec((NC, _RB, C), lambda i: (0, i, 0)),
            pl.BlockSpec((_RB, C), lambda i: (i, 0)),
            pl.BlockSpec((_RB, 1), lambda i: (i, 0)),
            pl.BlockSpec((1, C), lambda i: (0, 0)),
            pl.BlockSpec((C, C), lambda i: (0, 0)),
        ],
        out_specs=pl.BlockSpec((_RB, C), lambda i: (i, 0)),
        out_shape=jax.ShapeDtypeStruct((NP, C), jnp.float32),
    )(acc, p1, dis, b1, W2)


def _final_body(acc_ref, p2_ref, dis_ref, b2_ref, z_ref):
    agg = acc_ref[0] + acc_ref[1] - p2_ref[...]
    z_ref[...] = agg * dis_ref[...] + b2_ref[...]


def _tc_final(acc, p2, dis, b2):
    return pl.pallas_call(
        _final_body,
        grid=(NP // _RB,),
        in_specs=[
            pl.BlockSpec((NC, _RB, C), lambda i: (0, i, 0)),
            pl.BlockSpec((_RB, C), lambda i: (i, 0)),
            pl.BlockSpec((_RB, 1), lambda i: (i, 0)),
            pl.BlockSpec((1, C), lambda i: (0, 0)),
        ],
        out_specs=pl.BlockSpec((_RB, C), lambda i: (i, 0)),
        out_shape=jax.ShapeDtypeStruct((NP, C), jnp.float32),
    )(acc, p2, dis, b2)


def kernel(x, edge_index, W1, b1, W2, b2):
    src2 = edge_index[0].reshape(NW, EPW)
    dst2 = edge_index[1].reshape(NW, EPW)
    dst3 = dst2.reshape(NW, NCH, CHUNK)
    zeros = jnp.zeros((NP,), jnp.float32)
    ones = jnp.ones((CHUNK,), jnp.float32)
    x_p = jnp.pad(x, ((0, NP - N), (0, 0)))

    deg_p = _deg_kernel(dst3, zeros, ones)
    deg3 = deg_p.reshape(NC, NP, 1)
    p1, dis = _tc_prep(deg3, x_p, W1)
    acc1 = _prop_kernel(p1, src2, dst2)
    p2 = _tc_mid(acc1, p1, dis, b1.reshape(1, C), W2)
    acc2 = _prop_kernel(p2, src2, dst2)
    z = _tc_final(acc2, p2, dis, b2.reshape(1, C))
    return z[:N]
